# baseline (speedup 1.0000x reference)
import functools

import jax
import jax.numpy as jnp
from jax import lax
from jax.experimental import pallas as pl
from jax.experimental.pallas import tpu as pltpu
from jax.experimental.pallas import tpu_sc as plsc

VOCAB = 100000
D = 1024
B = 4
S = 4096
T = B * S

NC = 2
NS = 16
NW = NC * NS
TPW = T // NW
C = 8
NBUF = 2
G = 4
NCH = TPW // C
UNROLL = NBUF * G * 2 // 2
LANES = 16


def _body(ids_hbm, pos_hbm, wte_hbm, wpe_hbm, out_hbm,
          idx_tok, idx_pos,
          ra0, rb0, ra1, rb1, ro0, ro1,
          sg0, sg1, so0, so1):
    wid = lax.axis_index("s") * NC + lax.axis_index("c")
    base = wid * TPW
    pltpu.sync_copy(ids_hbm.at[pl.ds(base, TPW)], idx_tok)
    pltpu.sync_copy(pos_hbm.at[pl.ds(base, TPW)], idx_pos)

    RA = (ra0, ra1)
    RB = (rb0, rb1)
    RO = (ro0, ro1)
    SG = (sg0, sg1)
    SO = (so0, so1)

    def issue_gathers(ch, b):
        c0 = ch * C
        pltpu.async_copy(wte_hbm.at[idx_tok.at[pl.ds(c0, C)]], RA[b], SG[b])
        pltpu.async_copy(wpe_hbm.at[idx_pos.at[pl.ds(c0, C)]], RB[b], SG[b])

    for b in range(NBUF):
        issue_gathers(b, b)

    NI = NCH // UNROLL

    def it(i, carry):
        for k in range(UNROLL):
            b = k % NBUF
            gb = k // G
            slot = k % G
            ch = i * UNROLL + k

            pltpu.make_async_copy(wte_hbm.at[pl.ds(0, C)], RA[b], SG[b]).wait()
            pltpu.make_async_copy(wte_hbm.at[pl.ds(0, C)], RB[b], SG[b]).wait()

            if slot == 0:
                @pl.when(i > 0)
                def _wait_out(_gb=gb):
                    pltpu.make_async_copy(
                        RO[_gb], out_hbm.at[pl.ds(0, G * C)], SO[_gb]).wait()

            def row_body(r, c2, _b=b, _gb=gb, _slot=slot):
                for j in range(D // LANES):
                    sl = pl.ds(j * LANES, LANES)
                    RO[_gb][_slot * C + r, sl] = RA[_b][r, sl] + RB[_b][r, sl]
                return c2
            lax.fori_loop(0, C, row_body, 0)

            @pl.when(ch + NBUF < NCH)
            def _prefetch(_ch=ch, _b=b):
                issue_gathers(_ch + NBUF, _b)

            if slot == G - 1:
                first = ch - (G - 1)
                pltpu.async_copy(
                    RO[gb], out_hbm.at[pl.ds(base + first * C, G * C)], SO[gb])
        return carry

    lax.fori_loop(0, NI, it, 0)
    for gb in range(2):
        pltpu.make_async_copy(RO[gb], out_hbm.at[pl.ds(0, G * C)], SO[gb]).wait()


_embed_call = functools.partial(
    pl.kernel,
    out_type=jax.ShapeDtypeStruct((T, D), jnp.float32),
    mesh=plsc.VectorSubcoreMesh(core_axis_name="c", subcore_axis_name="s"),
    scratch_types=(
        [pltpu.VMEM((TPW,), jnp.int32)] * 2
        + [pltpu.VMEM((C, D), jnp.float32)] * (2 * NBUF)
        + [pltpu.VMEM((G * C, D), jnp.float32)] * 2
        + [pltpu.SemaphoreType.DMA] * 4
    ),
)(_body)


def kernel(input_ids, position_ids, wte, wpe):
    ids = input_ids.reshape(T).astype(jnp.int32)
    pos = position_ids.reshape(T).astype(jnp.int32)
    out = _embed_call(ids, pos, wte, wpe)
    return out.reshape(B, S, D)


# speedup vs baseline: 1.1298x; 1.1298x over previous
import functools

import jax
import jax.numpy as jnp
from jax import lax
from jax.experimental import pallas as pl
from jax.experimental.pallas import tpu as pltpu
from jax.experimental.pallas import tpu_sc as plsc

VOCAB = 100000
D = 1024
B = 4
S = 4096
T = B * S

NC = 2
NS = 16
NW = NC * NS
TPW = T // NW
C = 8
NBUF = 5
NCH = TPW // C
NI = NCH // NBUF
TAIL = NCH - NI * NBUF
LANES = 16


def _body(ids_hbm, pos_hbm, wte_hbm, wpe_hbm, out_hbm,
          idx_tok, idx_pos,
          ra0, rb0, ro0, ra1, rb1, ro1, ra2, rb2, ro2,
          ra3, rb3, ro3, ra4, rb4, ro4,
          sg0, sg1, sg2, sg3, sg4, so0, so1, so2, so3, so4):
    wid = lax.axis_index("s") * NC + lax.axis_index("c")
    base = wid * TPW
    pltpu.sync_copy(ids_hbm.at[pl.ds(base, TPW)], idx_tok)
    pltpu.sync_copy(pos_hbm.at[pl.ds(base, TPW)], idx_pos)

    RA = (ra0, ra1, ra2, ra3, ra4)
    RB = (rb0, rb1, rb2, rb3, rb4)
    RO = (ro0, ro1, ro2, ro3, ro4)
    SG = (sg0, sg1, sg2, sg3, sg4)
    SO = (so0, so1, so2, so3, so4)

    def issue_gathers(ch, b):
        c0 = ch * C
        pltpu.async_copy(wte_hbm.at[idx_tok.at[pl.ds(c0, C)]], RA[b], SG[b])
        pltpu.async_copy(wpe_hbm.at[idx_pos.at[pl.ds(c0, C)]], RB[b], SG[b])

    def wait_gathers(b):
        pltpu.make_async_copy(wte_hbm.at[pl.ds(0, C)], RA[b], SG[b]).wait()
        pltpu.make_async_copy(wte_hbm.at[pl.ds(0, C)], RB[b], SG[b]).wait()

    def wait_out(b):
        pltpu.make_async_copy(RO[b], out_hbm.at[pl.ds(0, C)], SO[b]).wait()

    def add_rows(b):
        def row_body(r, c2, _b=b):
            for j in range(D // LANES):
                sl = pl.ds(j * LANES, LANES)
                RO[_b][r, sl] = RA[_b][r, sl] + RB[_b][r, sl]
            return c2
        lax.fori_loop(0, C, row_body, 0)

    for b in range(NBUF):
        issue_gathers(b, b)

    def it(i, carry):
        for b in range(NBUF):
            ch = i * NBUF + b
            wait_gathers(b)

            @pl.when(i > 0)
            def _wait_out(_b=b):
                wait_out(_b)

            add_rows(b)

            @pl.when(ch + NBUF < NCH)
            def _prefetch(_ch=ch, _b=b):
                issue_gathers(_ch + NBUF, _b)

            pltpu.async_copy(RO[b], out_hbm.at[pl.ds(base + ch * C, C)], SO[b])
        return carry

    lax.fori_loop(0, NI, it, 0)

    for k in range(TAIL):
        ch = NI * NBUF + k
        wait_gathers(k)
        wait_out(k)
        add_rows(k)
        pltpu.async_copy(RO[k], out_hbm.at[pl.ds(base + ch * C, C)], SO[k])

    for b in range(NBUF):
        wait_out(b)


_embed_call = functools.partial(
    pl.kernel,
    out_type=jax.ShapeDtypeStruct((T, D), jnp.float32),
    mesh=plsc.VectorSubcoreMesh(core_axis_name="c", subcore_axis_name="s"),
    scratch_types=(
        [pltpu.VMEM((TPW,), jnp.int32)] * 2
        + [pltpu.VMEM((C, D), jnp.float32)] * (3 * NBUF)
        + [pltpu.SemaphoreType.DMA] * (2 * NBUF)
    ),
)(_body)


def kernel(input_ids, position_ids, wte, wpe):
    ids = input_ids.reshape(T).astype(jnp.int32)
    pos = position_ids.reshape(T).astype(jnp.int32)
    out = _embed_call(ids, pos, wte, wpe)
    return out.reshape(B, S, D)


# speedup vs baseline: 1.1863x; 1.0500x over previous
import functools

import jax
import jax.numpy as jnp
from jax import lax
from jax.experimental import pallas as pl
from jax.experimental.pallas import tpu as pltpu
from jax.experimental.pallas import tpu_sc as plsc

VOCAB = 100000
D = 1024
B = 4
S = 4096
T = B * S

NC = 2
NS = 16
NW = NC * NS
TPW = T // NW
C = 16
NBUF = 2
NCH = TPW // C
LANES = 16


def _body(ids_hbm, pos_hbm, wte_hbm, wpe_hbm, out_hbm,
          idx_tok, idx_pos,
          ra0, rb0, ro0, ra1, rb1, ro1,
          sg0, sg1, so0, so1):
    wid = lax.axis_index("s") * NC + lax.axis_index("c")
    base = wid * TPW
    pltpu.sync_copy(ids_hbm.at[pl.ds(base, TPW)], idx_tok)
    pltpu.sync_copy(pos_hbm.at[pl.ds(base, TPW)], idx_pos)

    RA = (ra0, ra1)
    RB = (rb0, rb1)
    RO = (ro0, ro1)
    SG = (sg0, sg1)
    SO = (so0, so1)

    H = C // 2

    def issue_gathers(ch, b):
        c0 = ch * C
        pltpu.async_copy(
            wte_hbm.at[idx_tok.at[pl.ds(c0, H)]], RA[b].at[pl.ds(0, H)], SG[b])
        pltpu.async_copy(
            wte_hbm.at[idx_tok.at[pl.ds(c0 + H, H)]], RA[b].at[pl.ds(H, H)],
            SG[b])
        pltpu.async_copy(
            wpe_hbm.at[idx_pos.at[pl.ds(c0, H)]], RB[b].at[pl.ds(0, H)], SG[b])
        pltpu.async_copy(
            wpe_hbm.at[idx_pos.at[pl.ds(c0 + H, H)]], RB[b].at[pl.ds(H, H)],
            SG[b])

    for b in range(NBUF):
        issue_gathers(b, b)

    NI = NCH // NBUF

    def it(i, carry):
        for b in range(NBUF):
            ch = i * NBUF + b
            pltpu.make_async_copy(wte_hbm.at[pl.ds(0, C)], RA[b], SG[b]).wait()
            pltpu.make_async_copy(wte_hbm.at[pl.ds(0, C)], RB[b], SG[b]).wait()

            @pl.when(i > 0)
            def _wait_out(_b=b):
                pltpu.make_async_copy(
                    RO[_b], out_hbm.at[pl.ds(0, C)], SO[_b]).wait()

            def row_body(r, c2, _b=b):
                for j in range(D // LANES):
                    sl = pl.ds(j * LANES, LANES)
                    RO[_b][r, sl] = RA[_b][r, sl] + RB[_b][r, sl]
                return c2
            lax.fori_loop(0, C, row_body, 0)

            @pl.when(i < NI - 1)
            def _prefetch(_ch=ch, _b=b):
                issue_gathers(_ch + NBUF, _b)

            pltpu.async_copy(RO[b], out_hbm.at[pl.ds(base + ch * C, C)], SO[b])
        return carry

    lax.fori_loop(0, NI, it, 0)
    for b in range(NBUF):
        pltpu.make_async_copy(RO[b], out_hbm.at[pl.ds(0, C)], SO[b]).wait()


_embed_call = functools.partial(
    pl.kernel,
    out_type=jax.ShapeDtypeStruct((T, D), jnp.float32),
    mesh=plsc.VectorSubcoreMesh(core_axis_name="c", subcore_axis_name="s"),
    scratch_types=(
        [pltpu.VMEM((TPW,), jnp.int32)] * 2
        + [pltpu.VMEM((C, D), jnp.float32)] * (3 * NBUF)
        + [pltpu.SemaphoreType.DMA] * (2 * NBUF)
    ),
)(_body)


def kernel(input_ids, position_ids, wte, wpe):
    ids = input_ids.reshape(T).astype(jnp.int32)
    pos = position_ids.reshape(T).astype(jnp.int32)
    out = _embed_call(ids, pos, wte, wpe)
    return out.reshape(B, S, D)
